# baseline (speedup 1.0000x reference)
import jax
import jax.numpy as jnp
from jax import lax
from jax.experimental import pallas as pl
from jax.experimental.pallas import tpu as pltpu
from jax.experimental.pallas import tpu_sc as plsc

ROWS = 16384
COLS = 1000
K = 200
L = 16
NW = 32
NGROUPS = 13
CBLK = 512
NCB = ROWS // CBLK
NTASKS = NGROUPS * NCB
MAX_TASKS_PER_W = (NTASKS + NW - 1) // NW


def _task_coords(t):
    g = t // NCB
    cb = t % NCB
    off = jnp.where(g < NGROUPS - 1, g * L, K - L)
    return off, cb * CBLK


NBUF = 3


def _xmask_kernel(xt_hbm, mask_hbm, out_hbm, mask_v, b0, b1, b2, g0, g1, g2, w0, w1, w2):
    wid = lax.axis_index("s") * 2 + lax.axis_index("c")
    pltpu.sync_copy(mask_hbm, mask_v)
    bufs = (b0, b1, b2)
    gsems = (g0, g1, g2)
    wsems = (w0, w1, w2)

    def gather_copy(t, b):
        off, c0 = _task_coords(t)
        idx = mask_v[pl.ds(off, L)]
        return pltpu.make_async_copy(
            xt_hbm.at[idx, pl.ds(c0, CBLK)], bufs[b], gsems[b]
        )

    def write_copy(t, b):
        off, c0 = _task_coords(t)
        return pltpu.make_async_copy(
            bufs[b], out_hbm.at[pl.ds(off, L), pl.ds(c0, CBLK)], wsems[b]
        )

    def guarded(t, fn):
        @pl.when(t < NTASKS)
        def _():
            fn()

    for k in range(min(NBUF, MAX_TASKS_PER_W)):
        t = wid + NW * k
        guarded(t, lambda t=t, b=k: gather_copy(t, b).start())
    for k in range(MAX_TASKS_PER_W):
        t = wid + NW * k
        b = k % NBUF
        guarded(t, lambda t=t, b=b: (gather_copy(t, b).wait(), write_copy(t, b).start()))
        kn = k + NBUF
        if kn < MAX_TASKS_PER_W:
            tn = wid + NW * kn
            bn = kn % NBUF
            guarded(tn, lambda t=tn, b=bn, tp=wid + NW * (kn - NBUF): (
                write_copy(tp, b).wait(), gather_copy(t, b).start()))
    for k in range(max(0, MAX_TASKS_PER_W - NBUF), MAX_TASKS_PER_W):
        t = wid + NW * k
        b = k % NBUF
        guarded(t, lambda t=t, b=b: write_copy(t, b).wait())

    return


def kernel(x, mask):
    mesh = plsc.VectorSubcoreMesh(core_axis_name="c", subcore_axis_name="s")
    run = pl.kernel(
        _xmask_kernel,
        mesh=mesh,
        out_type=jax.ShapeDtypeStruct((K, ROWS), jnp.float32),
        scratch_types=[
            pltpu.VMEM((K,), jnp.int32),
            pltpu.VMEM((L, CBLK), jnp.float32),
            pltpu.VMEM((L, CBLK), jnp.float32),
            pltpu.VMEM((L, CBLK), jnp.float32),
            pltpu.SemaphoreType.DMA,
            pltpu.SemaphoreType.DMA,
            pltpu.SemaphoreType.DMA,
            pltpu.SemaphoreType.DMA,
            pltpu.SemaphoreType.DMA,
            pltpu.SemaphoreType.DMA,
        ],
        compiler_params=pltpu.CompilerParams(
            needs_layout_passes=False, skip_device_barrier=True,
            disable_bounds_checks=True, disable_semaphore_checks=True
        ),
    )
    return run(x.T, mask).T


# speedup vs baseline: 3.0972x; 3.0972x over previous
import jax
import jax.numpy as jnp
from jax.experimental import pallas as pl
from jax.experimental.pallas import tpu as pltpu

ROWS = 16384
COLS = 1000
K = 200


def _tc_body(mask_ref, x_hbm, o_hbm, bufs, gsem, wsem):
    for j in range(K):
        m = mask_ref[j]
        pltpu.make_async_copy(x_hbm.at[pl.ds(m, 1), :], bufs.at[j], gsem).start()
    for j in range(K):
        m = mask_ref[j]
        pltpu.make_async_copy(x_hbm.at[pl.ds(m, 1), :], bufs.at[j], gsem).wait()
    for j in range(K):
        pltpu.make_async_copy(bufs.at[j], o_hbm.at[pl.ds(j, 1), :], wsem).start()
    for j in range(K):
        pltpu.make_async_copy(bufs.at[j], o_hbm.at[pl.ds(j, 1), :], wsem).wait()


def kernel(x, mask):
    xt = x.T
    grid_spec = pltpu.PrefetchScalarGridSpec(
        num_scalar_prefetch=1,
        grid=(1,),
        in_specs=[pl.BlockSpec(memory_space=pltpu.HBM)],
        out_specs=pl.BlockSpec(memory_space=pltpu.HBM),
        scratch_shapes=[
            pltpu.VMEM((K, 1, ROWS), jnp.float32),
            pltpu.SemaphoreType.DMA,
            pltpu.SemaphoreType.DMA,
        ],
    )
    outt = pl.pallas_call(
        _tc_body,
        grid_spec=grid_spec,
        out_shape=jax.ShapeDtypeStruct((K, ROWS), jnp.float32),
    )(mask, xt)
    return outt.T
